# baseline (speedup 1.0000x reference)
import functools

import numpy as np
import jax
import jax.numpy as jnp
from jax import lax
from jax.experimental import pallas as pl
from jax.experimental.pallas import tpu as pltpu
from jax.experimental.pallas import tpu_sc as plsc

_U = 10000
_I = 10000
_A = 500
_D = 64
_ES = 128

_PERM_U = np.random.default_rng(0).permutation(_U)
_PERM_I = np.random.default_rng(1).permutation(_I)


def _revmm_body(rf_ref, w1_ref, w2_ref, o1_ref, o2_ref):
    x = rf_ref[...]
    o1_ref[...] = jnp.dot(x, w1_ref[...], preferred_element_type=jnp.float32)
    o2_ref[...] = jnp.dot(x, w2_ref[...], preferred_element_type=jnp.float32)


def _rev_matmuls(rf, w1, w2):
    E = rf.shape[0]
    BE = 2000
    return pl.pallas_call(
        _revmm_body,
        grid=(E // BE,),
        in_specs=[
            pl.BlockSpec((BE, _ES), lambda i: (i, 0)),
            pl.BlockSpec((_ES, 4 * _D), lambda i: (0, 0)),
            pl.BlockSpec((_ES, _D), lambda i: (0, 0)),
        ],
        out_specs=[
            pl.BlockSpec((BE, 4 * _D), lambda i: (i, 0)),
            pl.BlockSpec((BE, _D), lambda i: (i, 0)),
        ],
        out_shape=[
            jax.ShapeDtypeStruct((E, 4 * _D), jnp.float32),
            jax.ShapeDtypeStruct((E, _D), jnp.float32),
        ],
    )(rf, w1, w2)


def _sentmm_body(sf_ref, w_ref, o_ref):
    o_ref[...] = jnp.dot(sf_ref[...], w_ref[...],
                         preferred_element_type=jnp.float32)


def _sent_matmul(sf, w):
    E = sf.shape[0]
    BE = 2000
    return pl.pallas_call(
        _sentmm_body,
        grid=(E // BE,),
        in_specs=[
            pl.BlockSpec((BE, _D), lambda i: (i, 0)),
            pl.BlockSpec((_D, _D), lambda i: (0, 0)),
        ],
        out_specs=pl.BlockSpec((BE, _D), lambda i: (i, 0)),
        out_shape=jax.ShapeDtypeStruct((E, _D), jnp.float32),
    )(sf, w)


def _contrast_body(x_ref, y_ref, yp_ref, w_ref, o_ref):
    px = jnp.dot(x_ref[...], w_ref[...], preferred_element_type=jnp.float32)
    s_pos = jnp.sum(px * y_ref[...], axis=1)
    s_neg = jnp.sum(px * yp_ref[...], axis=1)
    o_ref[0, 0] = jnp.sum(jax.nn.softplus(-s_pos) + jax.nn.softplus(s_neg))


def _contrast(x, y, w, perm):
    N = x.shape[0]
    yp = y[perm]
    return pl.pallas_call(
        _contrast_body,
        in_specs=[
            pl.BlockSpec((N, _D), lambda: (0, 0)),
            pl.BlockSpec((N, _D), lambda: (0, 0)),
            pl.BlockSpec((N, _D), lambda: (0, 0)),
            pl.BlockSpec((_D, _D), lambda: (0, 0)),
        ],
        out_specs=pl.BlockSpec((1, 1), lambda: (0, 0), memory_space=pltpu.SMEM),
        out_shape=jax.ShapeDtypeStruct((1, 1), jnp.float32),
    )(x, y, yp, w)[0, 0] / N


_NTILE = 16
_SEG_CH = 128


def _sc_segsum_batch(ops0, ops1, n_seg, W, serial=False):
    assert W == 128
    E = ops0[0][0].shape[0]
    nch = E // _SEG_CH
    nloc = (nch + _NTILE - 1) // _NTILE
    rows = -(-n_seg // (_NTILE * 8)) * 8
    npad = rows * _NTILE
    nfull, tail = divmod(rows, _SEG_CH)
    zeros = jnp.zeros((_SEG_CH, W), jnp.float32)
    nops = len(ops0) + len(ops1)
    mesh = plsc.VectorSubcoreMesh(core_axis_name="c", subcore_axis_name="s")

    @functools.partial(
        pl.kernel,
        mesh=mesh,
        out_type=[jax.ShapeDtypeStruct((npad, W), jnp.float32)] * nops,
        scratch_types=[
            pltpu.VMEM((_SEG_CH, W), jnp.float32),
            pltpu.VMEM((_SEG_CH,), jnp.int32),
            pltpu.VMEM_SHARED((npad, W), jnp.float32),
        ],
    )
    def k(*refs):
        args, rest = refs[:2 * nops], refs[2 * nops:]
        zz_h = rest[0]
        outs = rest[1:1 + nops]
        buf, idx, acc = rest[1 + nops:]
        c = lax.axis_index("c")
        s = lax.axis_index("s")
        r0 = s * rows

        def row_blocks():
            blocks = [(t * _SEG_CH, _SEG_CH) for t in range(nfull)]
            if tail:
                blocks.append((nfull * _SEG_CH, tail))
            return blocks

        def one_op(msg_h, dst_h, out_h):
            pltpu.sync_copy(zz_h, buf)
            for off, ln in row_blocks():
                pltpu.sync_copy(buf.at[pl.ds(0, ln)],
                                acc.at[pl.ds(r0 + off, ln)])
            plsc.subcore_barrier()

            def chunk(j):
                e0 = j * _SEG_CH
                pltpu.sync_copy(msg_h.at[pl.ds(e0, _SEG_CH)], buf)
                pltpu.sync_copy(dst_h.at[pl.ds(e0, _SEG_CH)], idx)
                pltpu.sync_copy(buf, acc.at[idx], add=True)

            if serial:
                def body(j, carry):
                    chunk(j)
                    return carry

                @pl.when(s == 0)
                def _():
                    lax.fori_loop(0, nch, body, 0)
            else:
                def body(jj, carry):
                    j = jj * _NTILE + s

                    @pl.when(j < nch)
                    def _():
                        chunk(j)
                    return carry

                lax.fori_loop(0, nloc, body, 0)
            plsc.subcore_barrier()
            for off, ln in row_blocks():
                pltpu.sync_copy(acc.at[pl.ds(r0 + off, ln)],
                                buf.at[pl.ds(0, ln)])
                pltpu.sync_copy(buf.at[pl.ds(0, ln)],
                                out_h.at[pl.ds(r0 + off, ln)])
            plsc.subcore_barrier()

        for core_id, core_ops in ((0, range(len(ops0))),
                                  (1, range(len(ops0), nops))):
            @pl.when(c == core_id)
            def _():
                for i in core_ops:
                    one_op(args[2 * i], args[2 * i + 1], outs[i])

    flat = []
    for msg, dst in ops0 + ops1:
        assert msg.shape == (E, W) and E % _SEG_CH == 0
        flat += [msg, dst]
    res = k(*flat, zeros)
    return [o[:n_seg] for o in res]


def _sc_gather_batch(row_ops, scalar_ops):
    E = (row_ops + scalar_ops)[0][1].shape[0]
    nw = 2 * _NTILE
    nq = 4
    grain = _SEG_CH * nw
    E_pad = -(-E // grain) * grain
    nch = E_pad // _SEG_CH
    nloc = nch // (nw * nq)
    mesh = plsc.VectorSubcoreMesh(core_axis_name="c", subcore_axis_name="s")
    out_type = ([jax.ShapeDtypeStruct((E_pad, 128), jnp.float32)]
                * len(row_ops)
                + [jax.ShapeDtypeStruct((E_pad,), jnp.float32)]
                * len(scalar_ops))

    @functools.partial(
        pl.kernel,
        mesh=mesh,
        out_type=out_type,
        scratch_types=[
            pltpu.VMEM((nq, _SEG_CH, 128), jnp.float32),
            pltpu.VMEM((nq, _SEG_CH), jnp.float32),
            pltpu.VMEM((nq, _SEG_CH), jnp.int32),
            pltpu.SemaphoreType.DMA,
            pltpu.SemaphoreType.DMA,
        ],
    )
    def k(*refs):
        nops = len(row_ops) + len(scalar_ops)
        ins, rest = refs[:2 * nops], refs[2 * nops:]
        outs = rest[:nops]
        rowbuf, sbuf, idx, semg, semw = rest[nops:]
        c = lax.axis_index("c")
        s = lax.axis_index("s")
        w = c * _NTILE + s

        def one_op(tab_h, idx_h, out_h, buf):
            def body(t, carry):
                e0s = [((t * nq + q) * nw + w) * _SEG_CH for q in range(nq)]
                for q in range(nq):
                    pltpu.sync_copy(idx_h.at[pl.ds(e0s[q], _SEG_CH)],
                                    idx.at[q])
                gh = [pltpu.async_copy(tab_h.at[idx.at[q]], buf.at[q], semg)
                      for q in range(nq)]
                wh = []
                for q in range(nq):
                    gh[q].wait()
                    wh.append(pltpu.async_copy(
                        buf.at[q], out_h.at[pl.ds(e0s[q], _SEG_CH)], semw))
                for h in wh:
                    h.wait()
                return carry

            lax.fori_loop(0, nloc, body, 0)

        for i in range(len(row_ops)):
            one_op(ins[2 * i], ins[2 * i + 1], outs[i], rowbuf)
        for i in range(len(row_ops), nops):
            one_op(ins[2 * i], ins[2 * i + 1], outs[i], sbuf)

    flat = []
    npad = E_pad - E
    for tab, idx in row_ops + scalar_ops:
        flat += [tab, jnp.pad(idx, (0, npad))]
    return [o[:E] for o in k(*flat)]


def kernel(feature, weight, w_contrast, aspect_feat, aspect_w, aspect_w_r,
           sentiment_w, sentiment_w_r, review_w, review_r_w, review_w2,
           review_r_w2, score_emb, score_r_emb, score2_emb, score_r2_emb,
           cau_a, cai_a, cau_u, cai_i, caucol, caucol_r, caicol, caicol_r,
           cur_u, cur_i, cir_u, cir_i, sentiment_feat_au, sentiment_feat_ai,
           review_feat, review_r_feat, au_src, au_dst, ai_src, ai_dst,
           uu_src, uu_dst, uu_aspect, ii_src, ii_dst, ii_aspect,
           rev_src, rev_dst, rev_score, revr_src, revr_dst, revr_score):
    fe_u, fe_i = feature[:_U], feature[_U:]
    fee_u, fee_i = weight[:_U], weight[_U:]

    asp_fe = aspect_feat @ aspect_w
    asp_fe1 = aspect_feat @ aspect_w_r

    r_au = _sent_matmul(sentiment_feat_au, sentiment_w)
    r_ai = _sent_matmul(sentiment_feat_ai, sentiment_w_r)

    pad = lambda x: jnp.concatenate([x, jnp.zeros_like(x)], axis=1)
    asp_fe_p = pad(asp_fe)
    asp_fe1_p = pad(asp_fe1)
    fe_u_p, fe_i_p = pad(fe_u), pad(fe_i)
    fee_u_p, fee_i_p = pad(fee_u), pad(fee_i)

    (g_au, g_ai, g_uufe, g_uuasp, g_iife, g_iiasp, g_fee_i, g_fee_u,
     c_au, c_ai, c_uu, c_ii, c_rev, c_revr) = _sc_gather_batch(
        [(asp_fe_p, au_src), (asp_fe_p, ai_src),
         (fe_u_p, uu_src), (asp_fe1_p, uu_aspect),
         (fe_i_p, ii_src), (asp_fe1_p, ii_aspect),
         (fee_i_p, rev_src), (fee_u_p, revr_src)],
        [(cau_a[:, 0], au_src), (cai_a[:, 0], ai_src),
         (caucol[:, 0], uu_src), (caicol[:, 0], ii_src),
         (cur_i[:, 0], rev_src), (cir_u[:, 0], revr_src)])

    r_rev, r2_rev = _rev_matmuls(review_feat, review_w, review_w2)
    r_revr, r2_revr = _rev_matmuls(review_r_feat, review_r_w, review_r_w2)
    s2_rev = score2_emb[rev_score]
    s2_revr = score_r2_emb[revr_score]

    msg_au = (g_au + pad(r_au)) * c_au[:, None]
    msg_ai = (g_ai + pad(r_ai)) * c_ai[:, None]
    msg_uu = (g_uufe + g_uuasp) * c_uu[:, None]
    msg_ii = (g_iife + g_iiasp) * c_ii[:, None]
    m2 = (g_fee_i + pad(r2_rev)) * pad(jax.nn.sigmoid(s2_rev)) * c_rev[:, None]
    m2r = (g_fee_u + pad(r2_revr)) * pad(jax.nn.sigmoid(s2_revr)) * c_revr[:, None]

    h_u, h1_u, u_rr, h_i, h2_i, i_rr = [
        o[:, :_D] for o in _sc_segsum_batch(
            [(msg_au, au_dst), (msg_uu, uu_dst), (m2, rev_dst)],
            [(msg_ai, ai_dst), (msg_ii, ii_dst), (m2r, revr_dst)],
            _U, 2 * _D)]
    u_rr = u_rr * cur_u
    i_rr = i_rr * cir_i

    CL1_u1 = h_u * cau_u
    CL1_u2 = h1_u * caucol_r
    CL1_i1 = h_i * cai_i
    CL1_i2 = h2_i * caicol_r

    closs = (_contrast(CL1_u1, CL1_u2, w_contrast, _PERM_U)
             + _contrast(CL1_u2, CL1_u1, w_contrast, _PERM_U)
             + _contrast(CL1_i1, CL1_i2, w_contrast, _PERM_I)
             + _contrast(CL1_i2, CL1_i1, w_contrast, _PERM_I)
             + _contrast(CL1_u1, CL1_u1, w_contrast, _PERM_U)
             + _contrast(CL1_i1, CL1_i1, w_contrast, _PERM_I))

    T_u = jnp.concatenate([CL1_u1, CL1_u2], axis=-1)
    T_i = jnp.concatenate([CL1_i1, CL1_i2], axis=-1)
    g_Ti, g_Tu = _sc_gather_batch([(T_i, rev_src), (T_u, revr_src)], [])

    sig = jax.nn.sigmoid(score_emb[rev_score])
    sigr = jax.nn.sigmoid(score_r_emb[revr_score])
    m1L = (g_Ti + r_rev[:, :128]) * sig[:, :128] * c_rev[:, None]
    m1R = (g_Ti + r_rev[:, 128:]) * sig[:, 128:] * c_rev[:, None]
    m1rL = (g_Tu + r_revr[:, :128]) * sigr[:, :128] * c_revr[:, None]
    m1rR = (g_Tu + r_revr[:, 128:]) * sigr[:, 128:] * c_revr[:, None]
    uaL, iaL, uaR, iaR = _sc_segsum_batch(
        [(m1L, rev_dst), (m1rL, revr_dst)],
        [(m1R, rev_dst), (m1rR, revr_dst)],
        _U, 2 * _D)
    ua = jnp.concatenate([uaL, uaR], axis=1) * cur_u
    ia = jnp.concatenate([iaL, iaR], axis=1) * cir_i

    return (closs, ua, ia, u_rr, i_rr)


# speedup vs baseline: 1.2898x; 1.2898x over previous
import functools

import numpy as np
import jax
import jax.numpy as jnp
from jax import lax
from jax.experimental import pallas as pl
from jax.experimental.pallas import tpu as pltpu
from jax.experimental.pallas import tpu_sc as plsc

_U = 10000
_I = 10000
_A = 500
_D = 64
_ES = 128

_PERM_U = np.random.default_rng(0).permutation(_U)
_PERM_I = np.random.default_rng(1).permutation(_I)


def _revmm_body(rf_ref, w1_ref, w2_ref, o1_ref, o2_ref):
    x = rf_ref[...]
    o1_ref[...] = jnp.dot(x, w1_ref[...], preferred_element_type=jnp.float32)
    o2_ref[...] = jnp.dot(x, w2_ref[...], preferred_element_type=jnp.float32)


def _rev_matmuls(rf, w1, w2):
    E = rf.shape[0]
    BE = 2000
    return pl.pallas_call(
        _revmm_body,
        grid=(E // BE,),
        in_specs=[
            pl.BlockSpec((BE, _ES), lambda i: (i, 0)),
            pl.BlockSpec((_ES, 4 * _D), lambda i: (0, 0)),
            pl.BlockSpec((_ES, _D), lambda i: (0, 0)),
        ],
        out_specs=[
            pl.BlockSpec((BE, 4 * _D), lambda i: (i, 0)),
            pl.BlockSpec((BE, _D), lambda i: (i, 0)),
        ],
        out_shape=[
            jax.ShapeDtypeStruct((E, 4 * _D), jnp.float32),
            jax.ShapeDtypeStruct((E, _D), jnp.float32),
        ],
    )(rf, w1, w2)


def _sentmm_body(sf_ref, w_ref, o_ref):
    o_ref[...] = jnp.dot(sf_ref[...], w_ref[...],
                         preferred_element_type=jnp.float32)


def _sent_matmul(sf, w):
    E = sf.shape[0]
    BE = 2000
    return pl.pallas_call(
        _sentmm_body,
        grid=(E // BE,),
        in_specs=[
            pl.BlockSpec((BE, _D), lambda i: (i, 0)),
            pl.BlockSpec((_D, _D), lambda i: (0, 0)),
        ],
        out_specs=pl.BlockSpec((BE, _D), lambda i: (i, 0)),
        out_shape=jax.ShapeDtypeStruct((E, _D), jnp.float32),
    )(sf, w)


def _contrast_body(x_ref, y_ref, yp_ref, w_ref, o_ref):
    px = jnp.dot(x_ref[...], w_ref[...], preferred_element_type=jnp.float32)
    s_pos = jnp.sum(px * y_ref[...], axis=1)
    s_neg = jnp.sum(px * yp_ref[...], axis=1)
    o_ref[0, 0] = jnp.sum(jax.nn.softplus(-s_pos) + jax.nn.softplus(s_neg))


def _contrast(x, y, w, perm):
    N = x.shape[0]
    yp = y[perm]
    return pl.pallas_call(
        _contrast_body,
        in_specs=[
            pl.BlockSpec((N, _D), lambda: (0, 0)),
            pl.BlockSpec((N, _D), lambda: (0, 0)),
            pl.BlockSpec((N, _D), lambda: (0, 0)),
            pl.BlockSpec((_D, _D), lambda: (0, 0)),
        ],
        out_specs=pl.BlockSpec((1, 1), lambda: (0, 0), memory_space=pltpu.SMEM),
        out_shape=jax.ShapeDtypeStruct((1, 1), jnp.float32),
    )(x, y, yp, w)[0, 0] / N


_NTILE = 16
_SEG_CH = 128


def _sc_segsum_batch(ops0, ops1, n_seg, W, serial=False):
    assert W == 128
    E = ops0[0][0].shape[0]
    nch = E // _SEG_CH
    nloc = (nch + _NTILE - 1) // _NTILE
    rows = -(-n_seg // (_NTILE * 8)) * 8
    npad = rows * _NTILE
    nfull, tail = divmod(rows, _SEG_CH)
    zeros = jnp.zeros((_SEG_CH, W), jnp.float32)
    nops = len(ops0) + len(ops1)
    mesh = plsc.VectorSubcoreMesh(core_axis_name="c", subcore_axis_name="s")

    @functools.partial(
        pl.kernel,
        mesh=mesh,
        out_type=[jax.ShapeDtypeStruct((npad, W), jnp.float32)] * nops,
        scratch_types=[
            pltpu.VMEM((_SEG_CH, W), jnp.float32),
            pltpu.VMEM((_SEG_CH,), jnp.int32),
            pltpu.VMEM_SHARED((npad, W), jnp.float32),
        ],
    )
    def k(*refs):
        args, rest = refs[:2 * nops], refs[2 * nops:]
        zz_h = rest[0]
        outs = rest[1:1 + nops]
        buf, idx, acc = rest[1 + nops:]
        c = lax.axis_index("c")
        s = lax.axis_index("s")
        r0 = s * rows

        def row_blocks():
            blocks = [(t * _SEG_CH, _SEG_CH) for t in range(nfull)]
            if tail:
                blocks.append((nfull * _SEG_CH, tail))
            return blocks

        def one_op(msg_h, dst_h, out_h):
            pltpu.sync_copy(zz_h, buf)
            for off, ln in row_blocks():
                pltpu.sync_copy(buf.at[pl.ds(0, ln)],
                                acc.at[pl.ds(r0 + off, ln)])
            plsc.subcore_barrier()

            def chunk(j):
                e0 = j * _SEG_CH
                pltpu.sync_copy(msg_h.at[pl.ds(e0, _SEG_CH)], buf)
                pltpu.sync_copy(dst_h.at[pl.ds(e0, _SEG_CH)], idx)
                pltpu.sync_copy(buf, acc.at[idx], add=True)

            if serial:
                def body(j, carry):
                    chunk(j)
                    return carry

                @pl.when(s == 0)
                def _():
                    lax.fori_loop(0, nch, body, 0)
            else:
                def body(jj, carry):
                    j = jj * _NTILE + s

                    @pl.when(j < nch)
                    def _():
                        chunk(j)
                    return carry

                lax.fori_loop(0, nloc, body, 0)
            plsc.subcore_barrier()
            for off, ln in row_blocks():
                pltpu.sync_copy(acc.at[pl.ds(r0 + off, ln)],
                                buf.at[pl.ds(0, ln)])
                pltpu.sync_copy(buf.at[pl.ds(0, ln)],
                                out_h.at[pl.ds(r0 + off, ln)])
            plsc.subcore_barrier()

        for core_id, core_ops in ((0, range(len(ops0))),
                                  (1, range(len(ops0), nops))):
            @pl.when(c == core_id)
            def _():
                for i in core_ops:
                    one_op(args[2 * i], args[2 * i + 1], outs[i])

    flat = []
    for msg, dst in ops0 + ops1:
        assert msg.shape == (E, W) and E % _SEG_CH == 0
        flat += [msg, dst]
    res = k(*flat, zeros)
    return [o[:n_seg] for o in res]


def _sc_gather_batch(row_ops, scalar_ops):
    E = (row_ops + scalar_ops)[0][1].shape[0]
    nw = 2 * _NTILE
    nq = 4
    nch = E // _SEG_CH
    kmax = -(-nch // nw)
    kfull = (nch - (nw - 1)) // nw
    nquad = kfull // nq
    mesh = plsc.VectorSubcoreMesh(core_axis_name="c", subcore_axis_name="s")
    out_type = ([jax.ShapeDtypeStruct((E, 128), jnp.float32)] * len(row_ops)
                + [jax.ShapeDtypeStruct((E,), jnp.float32)]
                * len(scalar_ops))

    @functools.partial(
        pl.kernel,
        mesh=mesh,
        out_type=out_type,
        scratch_types=[
            pltpu.VMEM((nq, _SEG_CH, 128), jnp.float32),
            pltpu.VMEM((nq, _SEG_CH), jnp.float32),
            pltpu.VMEM((nq, _SEG_CH), jnp.int32),
            pltpu.SemaphoreType.DMA,
            pltpu.SemaphoreType.DMA,
        ],
    )
    def k(*refs):
        nops = len(row_ops) + len(scalar_ops)
        ins, rest = refs[:2 * nops], refs[2 * nops:]
        outs = rest[:nops]
        rowbuf, sbuf, idx, semg, semw = rest[nops:]
        c = lax.axis_index("c")
        s = lax.axis_index("s")
        w = c * _NTILE + s

        def one_op(tab_h, idx_h, out_h, buf):
            def quad(t, carry):
                e0s = [((t * nq + q) * nw + w) * _SEG_CH for q in range(nq)]
                for q in range(nq):
                    pltpu.sync_copy(idx_h.at[pl.ds(e0s[q], _SEG_CH)],
                                    idx.at[q])
                gh = [pltpu.async_copy(tab_h.at[idx.at[q]], buf.at[q], semg)
                      for q in range(nq)]
                wh = []
                for q in range(nq):
                    gh[q].wait()
                    wh.append(pltpu.async_copy(
                        buf.at[q], out_h.at[pl.ds(e0s[q], _SEG_CH)], semw))
                for h in wh:
                    h.wait()
                return carry

            lax.fori_loop(0, nquad, quad, 0)
            for k in range(nquad * nq, kmax):
                j = k * nw + w

                @pl.when(j < nch)
                def _():
                    e0 = j * _SEG_CH
                    pltpu.sync_copy(idx_h.at[pl.ds(e0, _SEG_CH)], idx.at[0])
                    pltpu.async_copy(tab_h.at[idx.at[0]], buf.at[0],
                                     semg).wait()
                    pltpu.sync_copy(buf.at[0], out_h.at[pl.ds(e0, _SEG_CH)])

        for i in range(len(row_ops)):
            one_op(ins[2 * i], ins[2 * i + 1], outs[i], rowbuf)
        for i in range(len(row_ops), nops):
            one_op(ins[2 * i], ins[2 * i + 1], outs[i], sbuf)

    flat = []
    for tab, idx in row_ops + scalar_ops:
        flat += [tab, idx]
    return k(*flat)


def kernel(feature, weight, w_contrast, aspect_feat, aspect_w, aspect_w_r,
           sentiment_w, sentiment_w_r, review_w, review_r_w, review_w2,
           review_r_w2, score_emb, score_r_emb, score2_emb, score_r2_emb,
           cau_a, cai_a, cau_u, cai_i, caucol, caucol_r, caicol, caicol_r,
           cur_u, cur_i, cir_u, cir_i, sentiment_feat_au, sentiment_feat_ai,
           review_feat, review_r_feat, au_src, au_dst, ai_src, ai_dst,
           uu_src, uu_dst, uu_aspect, ii_src, ii_dst, ii_aspect,
           rev_src, rev_dst, rev_score, revr_src, revr_dst, revr_score):
    fe_u, fe_i = feature[:_U], feature[_U:]
    fee_u, fee_i = weight[:_U], weight[_U:]

    asp_fe = aspect_feat @ aspect_w
    asp_fe1 = aspect_feat @ aspect_w_r

    r_au = _sent_matmul(sentiment_feat_au, sentiment_w)
    r_ai = _sent_matmul(sentiment_feat_ai, sentiment_w_r)

    pad = lambda x: jnp.concatenate([x, jnp.zeros_like(x)], axis=1)
    asp_fe_p = pad(asp_fe)
    asp_fe1_p = pad(asp_fe1)
    fe_u_p, fe_i_p = pad(fe_u), pad(fe_i)
    fee_u_p, fee_i_p = pad(fee_u), pad(fee_i)

    (g_au, g_ai, g_uufe, g_uuasp, g_iife, g_iiasp, g_fee_i, g_fee_u,
     c_au, c_ai, c_uu, c_ii, c_rev, c_revr) = _sc_gather_batch(
        [(asp_fe_p, au_src), (asp_fe_p, ai_src),
         (fe_u_p, uu_src), (asp_fe1_p, uu_aspect),
         (fe_i_p, ii_src), (asp_fe1_p, ii_aspect),
         (fee_i_p, rev_src), (fee_u_p, revr_src)],
        [(cau_a[:, 0], au_src), (cai_a[:, 0], ai_src),
         (caucol[:, 0], uu_src), (caicol[:, 0], ii_src),
         (cur_i[:, 0], rev_src), (cir_u[:, 0], revr_src)])

    r_rev, r2_rev = _rev_matmuls(review_feat, review_w, review_w2)
    r_revr, r2_revr = _rev_matmuls(review_r_feat, review_r_w, review_r_w2)
    s2_rev = score2_emb[rev_score]
    s2_revr = score_r2_emb[revr_score]

    msg_au = (g_au + pad(r_au)) * c_au[:, None]
    msg_ai = (g_ai + pad(r_ai)) * c_ai[:, None]
    msg_uu = (g_uufe + g_uuasp) * c_uu[:, None]
    msg_ii = (g_iife + g_iiasp) * c_ii[:, None]
    m2 = (g_fee_i + pad(r2_rev)) * pad(jax.nn.sigmoid(s2_rev)) * c_rev[:, None]
    m2r = (g_fee_u + pad(r2_revr)) * pad(jax.nn.sigmoid(s2_revr)) * c_revr[:, None]

    h_u, h1_u, u_rr, h_i, h2_i, i_rr = [
        o[:, :_D] for o in _sc_segsum_batch(
            [(msg_au, au_dst), (msg_uu, uu_dst), (m2, rev_dst)],
            [(msg_ai, ai_dst), (msg_ii, ii_dst), (m2r, revr_dst)],
            _U, 2 * _D)]
    u_rr = u_rr * cur_u
    i_rr = i_rr * cir_i

    CL1_u1 = h_u * cau_u
    CL1_u2 = h1_u * caucol_r
    CL1_i1 = h_i * cai_i
    CL1_i2 = h2_i * caicol_r

    closs = (_contrast(CL1_u1, CL1_u2, w_contrast, _PERM_U)
             + _contrast(CL1_u2, CL1_u1, w_contrast, _PERM_U)
             + _contrast(CL1_i1, CL1_i2, w_contrast, _PERM_I)
             + _contrast(CL1_i2, CL1_i1, w_contrast, _PERM_I)
             + _contrast(CL1_u1, CL1_u1, w_contrast, _PERM_U)
             + _contrast(CL1_i1, CL1_i1, w_contrast, _PERM_I))

    T_u = jnp.concatenate([CL1_u1, CL1_u2], axis=-1)
    T_i = jnp.concatenate([CL1_i1, CL1_i2], axis=-1)
    g_Ti, g_Tu = _sc_gather_batch([(T_i, rev_src), (T_u, revr_src)], [])

    sig = jax.nn.sigmoid(score_emb[rev_score])
    sigr = jax.nn.sigmoid(score_r_emb[revr_score])
    m1L = (g_Ti + r_rev[:, :128]) * sig[:, :128] * c_rev[:, None]
    m1R = (g_Ti + r_rev[:, 128:]) * sig[:, 128:] * c_rev[:, None]
    m1rL = (g_Tu + r_revr[:, :128]) * sigr[:, :128] * c_revr[:, None]
    m1rR = (g_Tu + r_revr[:, 128:]) * sigr[:, 128:] * c_revr[:, None]
    uaL, iaL, uaR, iaR = _sc_segsum_batch(
        [(m1L, rev_dst), (m1rL, revr_dst)],
        [(m1R, rev_dst), (m1rR, revr_dst)],
        _U, 2 * _D)
    ua = jnp.concatenate([uaL, uaR], axis=1) * cur_u
    ia = jnp.concatenate([iaL, iaR], axis=1) * cir_i

    return (closs, ua, ia, u_rr, i_rr)


# speedup vs baseline: 1.2910x; 1.0009x over previous
import functools

import numpy as np
import jax
import jax.numpy as jnp
from jax import lax
from jax.experimental import pallas as pl
from jax.experimental.pallas import tpu as pltpu
from jax.experimental.pallas import tpu_sc as plsc

_U = 10000
_I = 10000
_A = 500
_D = 64
_ES = 128

_PERM_U = np.random.default_rng(0).permutation(_U)
_PERM_I = np.random.default_rng(1).permutation(_I)


def _revmm_body(rf_ref, w1_ref, w2_ref, o1_ref, o2_ref):
    x = rf_ref[...]
    o1_ref[...] = jnp.dot(x, w1_ref[...], preferred_element_type=jnp.float32)
    o2_ref[...] = jnp.dot(x, w2_ref[...], preferred_element_type=jnp.float32)


def _rev_matmuls(rf, w1, w2):
    E = rf.shape[0]
    BE = 2000
    return pl.pallas_call(
        _revmm_body,
        grid=(E // BE,),
        in_specs=[
            pl.BlockSpec((BE, _ES), lambda i: (i, 0)),
            pl.BlockSpec((_ES, 4 * _D), lambda i: (0, 0)),
            pl.BlockSpec((_ES, _D), lambda i: (0, 0)),
        ],
        out_specs=[
            pl.BlockSpec((BE, 4 * _D), lambda i: (i, 0)),
            pl.BlockSpec((BE, _D), lambda i: (i, 0)),
        ],
        out_shape=[
            jax.ShapeDtypeStruct((E, 4 * _D), jnp.float32),
            jax.ShapeDtypeStruct((E, _D), jnp.float32),
        ],
    )(rf, w1, w2)


def _sentmm_body(sf_ref, w_ref, o_ref):
    o_ref[...] = jnp.dot(sf_ref[...], w_ref[...],
                         preferred_element_type=jnp.float32)


def _sent_matmul(sf, w):
    E = sf.shape[0]
    BE = 2000
    return pl.pallas_call(
        _sentmm_body,
        grid=(E // BE,),
        in_specs=[
            pl.BlockSpec((BE, _D), lambda i: (i, 0)),
            pl.BlockSpec((_D, _D), lambda i: (0, 0)),
        ],
        out_specs=pl.BlockSpec((BE, _D), lambda i: (i, 0)),
        out_shape=jax.ShapeDtypeStruct((E, _D), jnp.float32),
    )(sf, w)


def _contrast_body(x_ref, y_ref, yp_ref, w_ref, o_ref):
    px = jnp.dot(x_ref[...], w_ref[...], preferred_element_type=jnp.float32)
    s_pos = jnp.sum(px * y_ref[...], axis=1)
    s_neg = jnp.sum(px * yp_ref[...], axis=1)
    o_ref[0, 0] = jnp.sum(jax.nn.softplus(-s_pos) + jax.nn.softplus(s_neg))


def _contrast(x, y, w, perm):
    N = x.shape[0]
    yp = y[perm]
    return pl.pallas_call(
        _contrast_body,
        in_specs=[
            pl.BlockSpec((N, _D), lambda: (0, 0)),
            pl.BlockSpec((N, _D), lambda: (0, 0)),
            pl.BlockSpec((N, _D), lambda: (0, 0)),
            pl.BlockSpec((_D, _D), lambda: (0, 0)),
        ],
        out_specs=pl.BlockSpec((1, 1), lambda: (0, 0), memory_space=pltpu.SMEM),
        out_shape=jax.ShapeDtypeStruct((1, 1), jnp.float32),
    )(x, y, yp, w)[0, 0] / N


_NTILE = 16
_SEG_CH = 128


def _sc_segsum_batch(ops0, ops1, n_seg, W):
    assert W == 128
    E = ops0[0][0].shape[0]
    nch = E // _SEG_CH
    nloc = (nch + _NTILE - 1) // _NTILE
    rows = -(-n_seg // (_NTILE * 8)) * 8
    npad = rows * _NTILE
    nfull, tail = divmod(rows, _SEG_CH)
    zeros = jnp.zeros((_SEG_CH, W), jnp.float32)
    nops = len(ops0) + len(ops1)
    mesh = plsc.VectorSubcoreMesh(core_axis_name="c", subcore_axis_name="s")

    @functools.partial(
        pl.kernel,
        mesh=mesh,
        out_type=[jax.ShapeDtypeStruct((npad, W), jnp.float32)] * nops,
        scratch_types=[
            pltpu.VMEM((_SEG_CH, W), jnp.float32),
            pltpu.VMEM((_SEG_CH,), jnp.int32),
            pltpu.VMEM_SHARED((npad, W), jnp.float32),
        ],
    )
    def k(*refs):
        args, rest = refs[:2 * nops], refs[2 * nops:]
        zz_h = rest[0]
        outs = rest[1:1 + nops]
        buf, idx, acc = rest[1 + nops:]
        c = lax.axis_index("c")
        s = lax.axis_index("s")
        r0 = s * rows

        def row_blocks():
            blocks = [(t * _SEG_CH, _SEG_CH) for t in range(nfull)]
            if tail:
                blocks.append((nfull * _SEG_CH, tail))
            return blocks

        def one_op(msg_h, dst_h, out_h):
            pltpu.sync_copy(zz_h, buf)
            for off, ln in row_blocks():
                pltpu.sync_copy(buf.at[pl.ds(0, ln)],
                                acc.at[pl.ds(r0 + off, ln)])
            plsc.subcore_barrier()

            def chunk(j):
                e0 = j * _SEG_CH
                pltpu.sync_copy(msg_h.at[pl.ds(e0, _SEG_CH)], buf)
                pltpu.sync_copy(dst_h.at[pl.ds(e0, _SEG_CH)], idx)
                pltpu.sync_copy(buf, acc.at[idx], add=True)

            def body(jj, carry):
                j = jj * _NTILE + s

                @pl.when(j < nch)
                def _():
                    chunk(j)
                return carry

            lax.fori_loop(0, nloc, body, 0)
            plsc.subcore_barrier()
            for off, ln in row_blocks():
                pltpu.sync_copy(acc.at[pl.ds(r0 + off, ln)],
                                buf.at[pl.ds(0, ln)])
                pltpu.sync_copy(buf.at[pl.ds(0, ln)],
                                out_h.at[pl.ds(r0 + off, ln)])
            plsc.subcore_barrier()

        for core_id, core_ops in ((0, range(len(ops0))),
                                  (1, range(len(ops0), nops))):
            @pl.when(c == core_id)
            def _():
                for i in core_ops:
                    one_op(args[2 * i], args[2 * i + 1], outs[i])

    flat = []
    for msg, dst in ops0 + ops1:
        assert msg.shape == (E, W) and E % _SEG_CH == 0
        flat += [msg, dst]
    res = k(*flat, zeros)
    return [o[:n_seg] for o in res]


def _sc_gather_batch(row_ops, scalar_ops):
    E = (row_ops + scalar_ops)[0][1].shape[0]
    nw = 2 * _NTILE
    nq = 4
    nch = E // _SEG_CH
    kmax = -(-nch // nw)
    kfull = (nch - (nw - 1)) // nw
    nquad = kfull // nq
    mesh = plsc.VectorSubcoreMesh(core_axis_name="c", subcore_axis_name="s")
    out_type = ([jax.ShapeDtypeStruct((E, 128), jnp.float32)] * len(row_ops)
                + [jax.ShapeDtypeStruct((E,), jnp.float32)]
                * len(scalar_ops))

    @functools.partial(
        pl.kernel,
        mesh=mesh,
        out_type=out_type,
        scratch_types=[
            pltpu.VMEM((nq, _SEG_CH, 128), jnp.float32),
            pltpu.VMEM((nq, _SEG_CH), jnp.float32),
            pltpu.VMEM((nq, _SEG_CH), jnp.int32),
            pltpu.SemaphoreType.DMA,
            pltpu.SemaphoreType.DMA,
        ],
    )
    def k(*refs):
        nops = len(row_ops) + len(scalar_ops)
        ins, rest = refs[:2 * nops], refs[2 * nops:]
        outs = rest[:nops]
        rowbuf, sbuf, idx, semg, semw = rest[nops:]
        c = lax.axis_index("c")
        s = lax.axis_index("s")
        w = c * _NTILE + s

        def one_op(tab_h, idx_h, out_h, buf):
            def quad(t, carry):
                e0s = [((t * nq + q) * nw + w) * _SEG_CH for q in range(nq)]
                for q in range(nq):
                    pltpu.sync_copy(idx_h.at[pl.ds(e0s[q], _SEG_CH)],
                                    idx.at[q])
                gh = [pltpu.async_copy(tab_h.at[idx.at[q]], buf.at[q], semg)
                      for q in range(nq)]
                wh = []
                for q in range(nq):
                    gh[q].wait()
                    wh.append(pltpu.async_copy(
                        buf.at[q], out_h.at[pl.ds(e0s[q], _SEG_CH)], semw))
                for h in wh:
                    h.wait()
                return carry

            lax.fori_loop(0, nquad, quad, 0)
            for k in range(nquad * nq, kmax):
                j = k * nw + w

                @pl.when(j < nch)
                def _():
                    e0 = j * _SEG_CH
                    pltpu.sync_copy(idx_h.at[pl.ds(e0, _SEG_CH)], idx.at[0])
                    pltpu.async_copy(tab_h.at[idx.at[0]], buf.at[0],
                                     semg).wait()
                    pltpu.sync_copy(buf.at[0], out_h.at[pl.ds(e0, _SEG_CH)])

        for i in range(len(row_ops)):
            one_op(ins[2 * i], ins[2 * i + 1], outs[i], rowbuf)
        for i in range(len(row_ops), nops):
            one_op(ins[2 * i], ins[2 * i + 1], outs[i], sbuf)

    flat = []
    for tab, idx in row_ops + scalar_ops:
        flat += [tab, idx]
    return k(*flat)


def kernel(feature, weight, w_contrast, aspect_feat, aspect_w, aspect_w_r,
           sentiment_w, sentiment_w_r, review_w, review_r_w, review_w2,
           review_r_w2, score_emb, score_r_emb, score2_emb, score_r2_emb,
           cau_a, cai_a, cau_u, cai_i, caucol, caucol_r, caicol, caicol_r,
           cur_u, cur_i, cir_u, cir_i, sentiment_feat_au, sentiment_feat_ai,
           review_feat, review_r_feat, au_src, au_dst, ai_src, ai_dst,
           uu_src, uu_dst, uu_aspect, ii_src, ii_dst, ii_aspect,
           rev_src, rev_dst, rev_score, revr_src, revr_dst, revr_score):
    fe_u, fe_i = feature[:_U], feature[_U:]
    fee_u, fee_i = weight[:_U], weight[_U:]

    asp_fe = aspect_feat @ aspect_w
    asp_fe1 = aspect_feat @ aspect_w_r

    r_au = _sent_matmul(sentiment_feat_au, sentiment_w)
    r_ai = _sent_matmul(sentiment_feat_ai, sentiment_w_r)

    pad = lambda x: jnp.concatenate([x, jnp.zeros_like(x)], axis=1)
    asp_fe_p = pad(asp_fe)
    asp_fe1_p = pad(asp_fe1)
    fe_u_p, fe_i_p = pad(fe_u), pad(fe_i)
    fee_u_p, fee_i_p = pad(fee_u), pad(fee_i)

    (g_au, g_ai, g_uufe, g_uuasp, g_iife, g_iiasp, g_fee_i, g_fee_u,
     c_au, c_ai, c_uu, c_ii, c_rev, c_revr) = _sc_gather_batch(
        [(asp_fe_p, au_src), (asp_fe_p, ai_src),
         (fe_u_p, uu_src), (asp_fe1_p, uu_aspect),
         (fe_i_p, ii_src), (asp_fe1_p, ii_aspect),
         (fee_i_p, rev_src), (fee_u_p, revr_src)],
        [(cau_a[:, 0], au_src), (cai_a[:, 0], ai_src),
         (caucol[:, 0], uu_src), (caicol[:, 0], ii_src),
         (cur_i[:, 0], rev_src), (cir_u[:, 0], revr_src)])

    r_rev, r2_rev = _rev_matmuls(review_feat, review_w, review_w2)
    r_revr, r2_revr = _rev_matmuls(review_r_feat, review_r_w, review_r_w2)
    s2_rev = score2_emb[rev_score]
    s2_revr = score_r2_emb[revr_score]

    msg_au = (g_au + pad(r_au)) * c_au[:, None]
    msg_ai = (g_ai + pad(r_ai)) * c_ai[:, None]
    msg_uu = (g_uufe + g_uuasp) * c_uu[:, None]
    msg_ii = (g_iife + g_iiasp) * c_ii[:, None]
    m2 = (g_fee_i + pad(r2_rev)) * pad(jax.nn.sigmoid(s2_rev)) * c_rev[:, None]
    m2r = (g_fee_u + pad(r2_revr)) * pad(jax.nn.sigmoid(s2_revr)) * c_revr[:, None]

    h_u, h1_u, u_rr, h_i, h2_i, i_rr = [
        o[:, :_D] for o in _sc_segsum_batch(
            [(msg_au, au_dst), (msg_uu, uu_dst), (m2, rev_dst)],
            [(msg_ai, ai_dst), (msg_ii, ii_dst), (m2r, revr_dst)],
            _U, 2 * _D)]
    u_rr = u_rr * cur_u
    i_rr = i_rr * cir_i

    CL1_u1 = h_u * cau_u
    CL1_u2 = h1_u * caucol_r
    CL1_i1 = h_i * cai_i
    CL1_i2 = h2_i * caicol_r

    closs = (_contrast(CL1_u1, CL1_u2, w_contrast, _PERM_U)
             + _contrast(CL1_u2, CL1_u1, w_contrast, _PERM_U)
             + _contrast(CL1_i1, CL1_i2, w_contrast, _PERM_I)
             + _contrast(CL1_i2, CL1_i1, w_contrast, _PERM_I)
             + _contrast(CL1_u1, CL1_u1, w_contrast, _PERM_U)
             + _contrast(CL1_i1, CL1_i1, w_contrast, _PERM_I))

    T_u = jnp.concatenate([CL1_u1, CL1_u2], axis=-1)
    T_i = jnp.concatenate([CL1_i1, CL1_i2], axis=-1)
    g_Ti, g_Tu = _sc_gather_batch([(T_i, rev_src), (T_u, revr_src)], [])

    sig = jax.nn.sigmoid(score_emb[rev_score])
    sigr = jax.nn.sigmoid(score_r_emb[revr_score])
    m1L = (g_Ti + r_rev[:, :128]) * sig[:, :128] * c_rev[:, None]
    m1R = (g_Ti + r_rev[:, 128:]) * sig[:, 128:] * c_rev[:, None]
    m1rL = (g_Tu + r_revr[:, :128]) * sigr[:, :128] * c_revr[:, None]
    m1rR = (g_Tu + r_revr[:, 128:]) * sigr[:, 128:] * c_revr[:, None]
    uaL, iaL, uaR, iaR = _sc_segsum_batch(
        [(m1L, rev_dst), (m1rL, revr_dst)],
        [(m1R, rev_dst), (m1rR, revr_dst)],
        _U, 2 * _D)
    ua = jnp.concatenate([uaL, uaR], axis=1) * cur_u
    ia = jnp.concatenate([iaL, iaR], axis=1) * cir_i

    return (closs, ua, ia, u_rr, i_rr)
